# Initial kernel scaffold; baseline (speedup 1.0000x reference)
import jax
import jax.numpy as jnp
from jax.experimental import pallas as pl


def kernel(know, table):
    raise NotImplementedError("write your pallas kernel here")


# speedup vs baseline: 1.3200x; 1.3200x over previous
import functools

import jax
import jax.numpy as jnp
from jax import lax
from jax.experimental import pallas as pl
from jax.experimental.pallas import tpu as pltpu
from jax.experimental.pallas import tpu_sc as plsc

KNOW_LEN = 8334
HIDDEN = 1024
NUM_EMB = 100008
BATCH = 4096
HIST = 50

NC = 2
NS = 16
NW = NC * NS

B = BATCH * HIST
B_PER_W = B // NW
CHUNK = 40
STEPS = B_PER_W // CHUNK
NITER = STEPS // 2


def _sc_gather(know_flat, table):
    mesh = plsc.VectorSubcoreMesh(
        core_axis_name="c", subcore_axis_name="s",
        num_cores=NC, num_subcores=NS)

    @functools.partial(
        pl.kernel,
        out_type=jax.ShapeDtypeStruct((B, HIDDEN), jnp.float32),
        mesh=mesh,
        scratch_types=[
            pltpu.VMEM((B_PER_W,), jnp.int32),
            pltpu.VMEM((CHUNK, HIDDEN), jnp.float32),
            pltpu.VMEM((CHUNK, HIDDEN), jnp.float32),
            pltpu.SemaphoreType.DMA,
            pltpu.SemaphoreType.DMA,
            pltpu.SemaphoreType.DMA,
            pltpu.SemaphoreType.DMA,
        ],
    )
    def k(idx_hbm, table_hbm, out_hbm, idx_v, rows0, rows1, g0, g1, s0, s1):
        wid = lax.axis_index("s") * NC + lax.axis_index("c")
        base = wid * B_PER_W
        slots = ((rows0, g0, s0), (rows1, g1, s1))

        pltpu.sync_copy(idx_hbm.at[pl.ds(base, B_PER_W)], idx_v)

        def idx_at(g):
            return idx_v.at[pl.ds(g * CHUNK, CHUNK)]

        def out_at(g):
            return out_hbm.at[pl.ds(base + g * CHUNK, CHUNK)]

        for b, (rows, gsem, _) in enumerate(slots):
            pltpu.async_copy(table_hbm.at[idx_at(b)], rows, gsem)

        def body(i, carry):
            for b, (rows, gsem, ssem) in enumerate(slots):
                g = 2 * i + b
                pltpu.make_async_copy(table_hbm.at[idx_at(g)], rows, gsem).wait()
                pltpu.async_copy(rows, out_at(g), ssem)
            @pl.when(i < NITER - 1)
            def _():
                for b, (rows, gsem, ssem) in enumerate(slots):
                    g = 2 * i + b
                    pltpu.make_async_copy(rows, out_at(g), ssem).wait()
                    pltpu.async_copy(table_hbm.at[idx_at(g + 2)], rows, gsem)
            return carry

        lax.fori_loop(0, NITER, body, 0)

        for b, (rows, _, ssem) in enumerate(slots):
            g = STEPS - 2 + b
            pltpu.make_async_copy(rows, out_at(g), ssem).wait()

    return k(know_flat, table)


def kernel(know, table):
    out = _sc_gather(know.reshape(-1), table)
    return out.reshape(BATCH, HIST, HIDDEN)


# speedup vs baseline: 1.3249x; 1.0037x over previous
import functools

import jax
import jax.numpy as jnp
from jax import lax
from jax.experimental import pallas as pl
from jax.experimental.pallas import tpu as pltpu
from jax.experimental.pallas import tpu_sc as plsc

KNOW_LEN = 8334
HIDDEN = 1024
NUM_EMB = 100008
BATCH = 4096
HIST = 50

NC = 2
NS = 16
NW = NC * NS

B = BATCH * HIST
B_PER_W = B // NW
CHUNK = 40
STEPS = B_PER_W // CHUNK
NBUF = 3
NITER = (STEPS - 1) // NBUF


def _sc_gather(know_flat, table):
    mesh = plsc.VectorSubcoreMesh(
        core_axis_name="c", subcore_axis_name="s",
        num_cores=NC, num_subcores=NS)

    @functools.partial(
        pl.kernel,
        out_type=jax.ShapeDtypeStruct((B, HIDDEN), jnp.float32),
        mesh=mesh,
        scratch_types=(
            [pltpu.VMEM((B_PER_W,), jnp.int32)]
            + [pltpu.VMEM((CHUNK, HIDDEN), jnp.float32)] * NBUF
            + [pltpu.SemaphoreType.DMA] * (2 * NBUF)
        ),
    )
    def k(idx_hbm, table_hbm, out_hbm, idx_v, *bufs):
        rows_bufs = bufs[:NBUF]
        gsems = bufs[NBUF:2 * NBUF]
        ssems = bufs[2 * NBUF:]
        wid = lax.axis_index("s") * NC + lax.axis_index("c")
        base = wid * B_PER_W
        slots = tuple(zip(rows_bufs, gsems, ssems))

        pltpu.sync_copy(idx_hbm.at[pl.ds(base, B_PER_W)], idx_v)

        def idx_at(g):
            return idx_v.at[pl.ds(g * CHUNK, CHUNK)]

        def out_at(g):
            return out_hbm.at[pl.ds(base + g * CHUNK, CHUNK)]

        for b, (rows, gsem, _) in enumerate(slots):
            pltpu.async_copy(table_hbm.at[idx_at(b)], rows, gsem)

        def body(i, carry):
            g0 = i * NBUF
            for b, (rows, gsem, ssem) in enumerate(slots):
                g = g0 + b
                pltpu.make_async_copy(table_hbm.at[idx_at(g)], rows, gsem).wait()
                pltpu.async_copy(rows, out_at(g), ssem)
            for b, (rows, gsem, ssem) in enumerate(slots):
                g = g0 + b

                @pl.when(g + NBUF < STEPS)
                def _(rows=rows, gsem=gsem, ssem=ssem, g=g):
                    pltpu.make_async_copy(rows, out_at(g), ssem).wait()
                    pltpu.async_copy(table_hbm.at[idx_at(g + NBUF)], rows, gsem)
            return carry

        lax.fori_loop(0, NITER, body, 0)

        tail = NITER * NBUF
        for b in range(STEPS - tail):
            rows, gsem, ssem = slots[b]
            g = tail + b
            pltpu.make_async_copy(table_hbm.at[idx_at(g)], rows, gsem).wait()
            pltpu.async_copy(rows, out_at(g), ssem)
            pltpu.make_async_copy(rows, out_at(g), ssem).wait()
        for b in range(STEPS - tail, NBUF):
            rows, _, ssem = slots[b]
            g = tail - NBUF + b
            pltpu.make_async_copy(rows, out_at(g), ssem).wait()

    return k(know_flat, table)


def kernel(know, table):
    out = _sc_gather(know.reshape(-1), table)
    return out.reshape(BATCH, HIST, HIDDEN)


# speedup vs baseline: 4.0227x; 3.0362x over previous
import functools

import jax
import jax.numpy as jnp
from jax import lax
from jax.experimental import pallas as pl
from jax.experimental.pallas import tpu as pltpu
from jax.experimental.pallas import tpu_sc as plsc

KNOW_LEN = 8334
HIDDEN = 1024
NUM_EMB = 100008
BATCH = 4096
HIST = 50

NC = 2
NS = 16
NW = NC * NS

B = BATCH * HIST
B_PER_W = B // NW
CHUNK = 40
STEPS = B_PER_W // CHUNK
NBUF = 3
NITER = (STEPS - 1) // NBUF


def _sc_gather(know_flat, table):
    mesh = plsc.VectorSubcoreMesh(
        core_axis_name="c", subcore_axis_name="s",
        num_cores=NC, num_subcores=NS)

    @functools.partial(
        pl.kernel,
        out_type=jax.ShapeDtypeStruct((B, HIDDEN), jnp.float32),
        mesh=mesh,
        scratch_types=(
            [pltpu.VMEM((B_PER_W,), jnp.int32)]
            + [pltpu.VMEM((CHUNK, HIDDEN), jnp.float32)] * NBUF
            + [pltpu.SemaphoreType.DMA] * (2 * NBUF)
        ),
    )
    def k(idx_hbm, table_hbm, out_hbm, idx_v, *bufs):
        rows_bufs = bufs[:NBUF]
        gsems = bufs[NBUF:2 * NBUF]
        ssems = bufs[2 * NBUF:]
        wid = lax.axis_index("s") * NC + lax.axis_index("c")
        base = wid * B_PER_W
        slots = tuple(zip(rows_bufs, gsems, ssems))

        pltpu.sync_copy(idx_hbm.at[pl.ds(base, B_PER_W)], idx_v)

        def idx_at(g):
            return idx_v.at[pl.ds(g * CHUNK, CHUNK)]

        def out_at(g):
            return out_hbm.at[pl.ds(base + g * CHUNK, CHUNK)]

        for b, (rows, gsem, _) in enumerate(slots):
            pltpu.async_copy(table_hbm.at[idx_at(b)], rows, gsem)

        def body(i, carry):
            g0 = i * NBUF
            for b, (rows, gsem, ssem) in enumerate(slots):
                g = g0 + b
                pltpu.make_async_copy(table_hbm.at[idx_at(g)], rows, gsem).wait()
                pltpu.async_copy(rows, out_at(g), ssem)
            for b, (rows, gsem, ssem) in enumerate(slots):
                g = g0 + b

                @pl.when(g + NBUF < STEPS)
                def _(rows=rows, gsem=gsem, ssem=ssem, g=g):
                    pltpu.make_async_copy(rows, out_at(g), ssem).wait()
                    pltpu.async_copy(table_hbm.at[idx_at(g + NBUF)], rows, gsem)
            return carry

        lax.fori_loop(0, NITER, body, 0)

        tail = NITER * NBUF
        for b in range(STEPS - tail):
            rows, gsem, ssem = slots[b]
            g = tail + b
            pltpu.make_async_copy(table_hbm.at[idx_at(g)], rows, gsem).wait()
            pltpu.async_copy(rows, out_at(g), ssem)
            pltpu.make_async_copy(rows, out_at(g), ssem).wait()
        for b in range(STEPS - tail, NBUF):
            rows, _, ssem = slots[b]
            g = tail - NBUF + b
            pltpu.make_async_copy(rows, out_at(g), ssem).wait()

    return k(know_flat, table)


def kernel(know, table):
    out = _sc_gather(know.T.reshape(-1), table)
    return out.reshape(HIST, BATCH, HIDDEN).transpose(1, 0, 2)
